# baseline (speedup 1.0000x reference)
import functools

import numpy as np
import jax
import jax.numpy as jnp
from jax import lax
from jax.experimental import pallas as pl
from jax.experimental.pallas import tpu as pltpu
from jax.experimental.pallas import tpu_sc as plsc

NUM_FIELDS = 26
EMBED_DIM = 16
BATCH = 16384
FIELD_SIZE = 100000
_OFFSETS_NP = (np.arange(NUM_FIELDS) * FIELD_SIZE).astype(np.int32)
NUM_PAIRS = NUM_FIELDS * (NUM_FIELDS - 1) // 2

FPAD = 32
ROWS32 = BATCH * FPAD
TROWS = BATCH // 8
LTILES = FPAD * EMBED_DIM // 128

_iu, _ju = np.triu_indices(NUM_FIELDS, k=1)
_ref_pos = {(int(i), int(j)): p for p, (i, j) in enumerate(zip(_iu, _ju))}
_PERM = np.array(
    [_ref_pos[(i, i + k)] for k in range(1, NUM_FIELDS) for i in range(NUM_FIELDS - k)],
    dtype=np.int32,
)

_NC, _NS = 2, 16
_NW = _NC * _NS
_ROWS_PER_W = ROWS32 // _NW
_CHUNK = 4096
_NCHUNK = _ROWS_PER_W // _CHUNK
_TR_PER_CHUNK = _CHUNK // (8 * FPAD)

_sc_mesh = plsc.VectorSubcoreMesh(core_axis_name="c", subcore_axis_name="s")


@functools.partial(
    pl.kernel,
    mesh=_sc_mesh,
    out_type=jax.ShapeDtypeStruct((ROWS32, EMBED_DIM), jnp.float32),
    scratch_types=[
        pltpu.VMEM((_CHUNK,), jnp.int32),
        pltpu.VMEM((_CHUNK, EMBED_DIM), jnp.float32),
        pltpu.SemaphoreType.DMA,
    ],
    compiler_params=pltpu.CompilerParams(use_tc_tiling_on_sc=False),
)
def _sc_gather(idx_hbm, table_hbm, out_hbm, idx_v, rows_v, sem):
    wid = lax.axis_index("s") * _NC + lax.axis_index("c")
    base = wid * _ROWS_PER_W
    for c in range(_NCHUNK):
        off = base + c * _CHUNK
        pltpu.sync_copy(idx_hbm.at[pl.ds(off, _CHUNK)], idx_v)
        pltpu.async_copy(table_hbm.at[idx_v], rows_v, sem).wait()
        pltpu.sync_copy(rows_v, out_hbm.at[pl.ds(off, _CHUNK)])


_BB = 512
_GRID = BATCH // _BB
_BT = _BB // 8


def _tc_body(emb_ref, w1t_ref, b1_ref, w2t_ref, b2_ref, wot_ref, bo_ref, out_ref):
    cols = []
    for c in range(LTILES):
        vc = emb_ref[:, c, :, :].reshape(_BB, 128)
        cols.append(vc.T)
    et = jnp.concatenate(cols, axis=0)
    parts = []
    for k in range(1, NUM_FIELDS):
        n = NUM_FIELDS - k
        a = et[: n * EMBED_DIM, :]
        b = et[k * EMBED_DIM : (k + n) * EMBED_DIM, :]
        prod = (a * b).reshape(n, EMBED_DIM, _BB)
        parts.append(jnp.sum(prod, axis=1))
    hT = jnp.concatenate(parts, axis=0)
    z1 = jnp.dot(w1t_ref[...], hT, preferred_element_type=jnp.float32)
    h1 = jnp.maximum(z1 + b1_ref[...], 0.0)
    z2 = jnp.dot(w2t_ref[...], h1, preferred_element_type=jnp.float32)
    h2 = jnp.maximum(z2 + b2_ref[...], 0.0)
    o = jnp.dot(wot_ref[...], h2, preferred_element_type=jnp.float32) + bo_ref[...]
    out_ref[...] = jax.nn.sigmoid(o)


_tc_call = pl.pallas_call(
    _tc_body,
    grid=(_GRID,),
    in_specs=[
        pl.BlockSpec((_BT, LTILES, 8, 128), lambda i: (i, 0, 0, 0)),
        pl.BlockSpec((512, NUM_PAIRS), lambda i: (0, 0)),
        pl.BlockSpec((512, 1), lambda i: (0, 0)),
        pl.BlockSpec((256, 512), lambda i: (0, 0)),
        pl.BlockSpec((256, 1), lambda i: (0, 0)),
        pl.BlockSpec((1, 256), lambda i: (0, 0)),
        pl.BlockSpec((1, 1), lambda i: (0, 0)),
    ],
    out_specs=pl.BlockSpec((1, _BB), lambda i: (0, i)),
    out_shape=jax.ShapeDtypeStruct((1, BATCH), jnp.float32),
)


def kernel(x, table, W1, b1, W2, b2, Wout, bout):
    offsets = jnp.asarray(_OFFSETS_NP)
    idx = x + offsets[None, :]
    idx32 = jnp.concatenate(
        [idx, jnp.zeros((BATCH, FPAD - NUM_FIELDS), jnp.int32)], axis=1
    )
    idx_ord = (
        idx32.reshape(TROWS, 8, LTILES, 8).transpose(0, 2, 1, 3).reshape(ROWS32)
    )
    emb4 = _sc_gather(idx_ord, table).reshape(TROWS, LTILES, 8, 128)
    w1t = W1[jnp.asarray(_PERM), :].T
    out = _tc_call(
        emb4,
        w1t,
        b1.reshape(512, 1),
        W2.T,
        b2.reshape(256, 1),
        Wout.T,
        bout.reshape(1, 1),
    )
    return out.reshape(BATCH, 1)


# speedup vs baseline: 6.2557x; 6.2557x over previous
import functools

import numpy as np
import jax
import jax.numpy as jnp
from jax import lax
from jax.experimental import pallas as pl
from jax.experimental.pallas import tpu as pltpu
from jax.experimental.pallas import tpu_sc as plsc

NUM_FIELDS = 26
EMBED_DIM = 16
BATCH = 16384
FIELD_SIZE = 100000
_OFFSETS_NP = (np.arange(NUM_FIELDS) * FIELD_SIZE).astype(np.int32)
NUM_PAIRS = NUM_FIELDS * (NUM_FIELDS - 1) // 2

FPAD = 32
ROWS32 = BATCH * FPAD
LTILES = FPAD * EMBED_DIM // 128

_iu, _ju = np.triu_indices(NUM_FIELDS, k=1)
_ref_pos = {(int(i), int(j)): p for p, (i, j) in enumerate(zip(_iu, _ju))}
_PERM = np.array(
    [_ref_pos[(i, i + k)] for k in range(1, NUM_FIELDS) for i in range(NUM_FIELDS - k)],
    dtype=np.int32,
)

_NC, _NS = 2, 16
_NW = _NC * _NS
_CHUNK = 2048

_sc_mesh = plsc.VectorSubcoreMesh(core_axis_name="c", subcore_axis_name="s")


def _make_sc_gather(nrows):
    rows_per_w = nrows // _NW
    nchunk = rows_per_w // _CHUNK

    @functools.partial(
        pl.kernel,
        mesh=_sc_mesh,
        out_type=jax.ShapeDtypeStruct((nrows, EMBED_DIM), jnp.float32),
        scratch_types=[
            pltpu.VMEM((rows_per_w,), jnp.int32),
            pltpu.VMEM((_CHUNK, EMBED_DIM), jnp.float32),
            pltpu.VMEM((_CHUNK, EMBED_DIM), jnp.float32),
            pltpu.SemaphoreType.DMA,
            pltpu.SemaphoreType.DMA,
        ],
        compiler_params=pltpu.CompilerParams(use_tc_tiling_on_sc=False),
    )
    def gather(idx_hbm, table_hbm, out_hbm, idx_v, rows_a, rows_b, sem_a, sem_b):
        wid = lax.axis_index("s") * _NC + lax.axis_index("c")
        base = wid * rows_per_w
        pltpu.sync_copy(idx_hbm.at[pl.ds(base, rows_per_w)], idx_v)
        bufs = (rows_a, rows_b)
        sems = (sem_a, sem_b)
        handles = [None, None]
        handles[0] = pltpu.async_copy(
            table_hbm.at[idx_v.at[pl.ds(0, _CHUNK)]], bufs[0], sems[0]
        )
        for c in range(nchunk):
            nxt = c + 1
            if nxt < nchunk:
                handles[nxt % 2] = pltpu.async_copy(
                    table_hbm.at[idx_v.at[pl.ds(nxt * _CHUNK, _CHUNK)]],
                    bufs[nxt % 2],
                    sems[nxt % 2],
                )
            handles[c % 2].wait()
            pltpu.sync_copy(bufs[c % 2], out_hbm.at[pl.ds(base + c * _CHUNK, _CHUNK)])

    return gather


_NSPLIT = 2
_sc_gather_half = _make_sc_gather(ROWS32 // _NSPLIT)


TABLE_ROWS = 100000 * NUM_FIELDS
_PK_COLS = 32768
_PK_GRID = -(-TABLE_ROWS // _PK_COLS)
_PK_STRIP = _PK_COLS // 8
TABLE_ROWS_PAD = _PK_GRID * _PK_COLS


def _pack_body(tT_ref, out_ref):
    v = tT_ref[...]
    stacked = jnp.concatenate(
        [v[:, j * _PK_STRIP : (j + 1) * _PK_STRIP] for j in range(8)], axis=0
    )
    out_ref[...] = stacked.T


def _remap_rows(r):
    m = r // _PK_COLS
    j = (r // _PK_STRIP) % 8
    c = r % _PK_STRIP
    return m * _PK_COLS + c * 8 + j


_pack_call = pl.pallas_call(
    _pack_body,
    grid=(_PK_GRID,),
    in_specs=[pl.BlockSpec((EMBED_DIM, _PK_COLS), lambda i: (0, i))],
    out_specs=pl.BlockSpec((_PK_STRIP, 128), lambda i: (i, 0)),
    out_shape=jax.ShapeDtypeStruct((TABLE_ROWS_PAD // 8, 128), jnp.float32),
)


_BB = 512


def _tc_body(emb_ref, w1t_ref, b1_ref, w2t_ref, b2_ref, wot_ref, bo_ref, out_ref):
    cols = []
    for c in range(LTILES):
        vc = jnp.concatenate(
            [emb_ref[:, c : c + 1, :], emb_ref[:, 4 + c : 5 + c, :]], axis=1
        ).reshape(_BB, 128)
        cols.append(vc.T)
    et = jnp.concatenate(cols, axis=0)
    parts = []
    for k in range(1, NUM_FIELDS):
        n = NUM_FIELDS - k
        a = et[: n * EMBED_DIM, :]
        b = et[k * EMBED_DIM : (k + n) * EMBED_DIM, :]
        prod = (a * b).reshape(n, EMBED_DIM, _BB)
        parts.append(jnp.sum(prod, axis=1))
    hT = jnp.concatenate(parts, axis=0)
    z1 = jnp.dot(w1t_ref[...], hT, preferred_element_type=jnp.float32)
    h1 = jnp.maximum(z1 + b1_ref[...], 0.0)
    z2 = jnp.dot(w2t_ref[...], h1, preferred_element_type=jnp.float32)
    h2 = jnp.maximum(z2 + b2_ref[...], 0.0)
    o = jnp.dot(wot_ref[...], h2, preferred_element_type=jnp.float32) + bo_ref[...]
    out_ref[...] = jax.nn.sigmoid(o)


_BHALF = BATCH // _NSPLIT


_tc_call = pl.pallas_call(
    _tc_body,
    grid=(_BHALF // _BB,),
    in_specs=[
        pl.BlockSpec((_BB // 2, 8, 128), lambda i: (i, 0, 0)),
        pl.BlockSpec((512, NUM_PAIRS), lambda i: (0, 0)),
        pl.BlockSpec((512, 1), lambda i: (0, 0)),
        pl.BlockSpec((256, 512), lambda i: (0, 0)),
        pl.BlockSpec((256, 1), lambda i: (0, 0)),
        pl.BlockSpec((1, 256), lambda i: (0, 0)),
        pl.BlockSpec((1, 1), lambda i: (0, 0)),
    ],
    out_specs=pl.BlockSpec((1, _BB), lambda i: (0, i)),
    out_shape=jax.ShapeDtypeStruct((1, _BHALF), jnp.float32),
)


def kernel(x, table, W1, b1, W2, b2, Wout, bout):
    offsets = jnp.asarray(_OFFSETS_NP)
    idx = x + offsets[None, :]
    idx32 = jnp.concatenate(
        [idx] + [idx[:, NUM_FIELDS - 1 :]] * (FPAD - NUM_FIELDS), axis=1
    )
    idx_ord = _remap_rows(idx32.reshape(ROWS32))
    packed = _pack_call(table.T)
    table_lin = packed.reshape(TABLE_ROWS_PAD, EMBED_DIM)
    w1t = W1[jnp.asarray(_PERM), :].T
    w2t = W2.T
    wot = Wout.T
    rows_half = ROWS32 // _NSPLIT
    outs = []
    for h in range(_NSPLIT):
        idx_h = lax.slice(idx_ord, (h * rows_half,), ((h + 1) * rows_half,))
        emb4 = _sc_gather_half(idx_h, table_lin).reshape(
            rows_half * EMBED_DIM // 1024, 8, 128
        )
        outs.append(
            _tc_call(
                emb4,
                w1t,
                b1.reshape(512, 1),
                w2t,
                b2.reshape(256, 1),
                wot,
                bout.reshape(1, 1),
            )
        )
    return jnp.concatenate(outs, axis=1).reshape(BATCH, 1)
